# baseline (speedup 1.0000x reference)
import functools

import jax
import jax.numpy as jnp
from jax import lax
from jax.experimental import pallas as pl
from jax.experimental.pallas import tpu as pltpu
from jax.experimental.pallas import tpu_sc as plsc

N_NODES = 10000
N_EDGES = 320000
D = 128

NC = 2
NS = 16
NW = NC * NS
CHUNK = 200
NCHUNK = 50
EDGES_PER_TILE = CHUNK * NCHUNK
N_PAD = 10240
ROWS_PER_TILE = N_PAD // NS

_BLK = 2000


def _mm_relu_body(x_ref, w_ref, b_ref, o_ref):
    z = jnp.dot(x_ref[...], w_ref[...], preferred_element_type=jnp.float32)
    o_ref[...] = jnp.maximum(z + b_ref[...], 0.0).astype(jnp.bfloat16)


def _mm_relu(x, w, b):
    n = x.shape[0]
    return pl.pallas_call(
        _mm_relu_body,
        grid=(n // _BLK,),
        in_specs=[
            pl.BlockSpec((_BLK, D), lambda i: (i, 0)),
            pl.BlockSpec((D, D), lambda i: (0, 0)),
            pl.BlockSpec((1, D), lambda i: (0, 0)),
        ],
        out_specs=pl.BlockSpec((_BLK, D), lambda i: (i, 0)),
        out_shape=jax.ShapeDtypeStruct((n, D), jnp.bfloat16),
    )(x, w, b.reshape(1, D))


def _final_mm_body(p0_ref, p1_ref, w_ref, b_ref, o_ref):
    agg = (p0_ref[0].astype(jnp.float32) + p1_ref[0].astype(jnp.float32))
    o_ref[...] = (
        jnp.dot(agg, w_ref[...], preferred_element_type=jnp.float32)
        + b_ref[...])


def _final_mm(partials, w, b):
    n = N_NODES
    return pl.pallas_call(
        _final_mm_body,
        grid=(n // _BLK,),
        in_specs=[
            pl.BlockSpec((1, _BLK, D), lambda i: (0, i, 0)),
            pl.BlockSpec((1, _BLK, D), lambda i: (1, i, 0)),
            pl.BlockSpec((D, D), lambda i: (0, 0)),
            pl.BlockSpec((1, D), lambda i: (0, 0)),
        ],
        out_specs=pl.BlockSpec((_BLK, D), lambda i: (i, 0)),
        out_shape=jax.ShapeDtypeStruct((n, D), jnp.float32),
    )(partials, partials, w, b.reshape(1, D))


def _sc_edge_body(edges_hbm, z_hbm, zeros_hbm, out_hbm,
                  src_all, dst_all, rows0, rows1, z_sh, agg_sh, sem0, sem1):
    cid = lax.axis_index("c")
    sid = lax.axis_index("s")

    row_base = sid * ROWS_PER_TILE
    z_rows = N_NODES // NS
    with jax.named_scope("init_zero_idx"):
        pltpu.sync_copy(zeros_hbm, agg_sh.at[pl.ds(row_base, ROWS_PER_TILE)])
        pltpu.sync_copy(z_hbm.at[pl.ds(sid * z_rows, z_rows)],
                        z_sh.at[pl.ds(sid * z_rows, z_rows)])

        wid = cid * NS + sid
        ebase = wid * EDGES_PER_TILE
        pltpu.sync_copy(edges_hbm.at[pl.ds(ebase, EDGES_PER_TILE)], src_all)
        pltpu.sync_copy(edges_hbm.at[pl.ds(N_EDGES + ebase, EDGES_PER_TILE)],
                        dst_all)
        plsc.subcore_barrier()

    bufs = (rows0, rows1)
    sems = (sem0, sem1)

    def src_at(i):
        return src_all.at[pl.ds(i * CHUNK, CHUNK)]

    def dst_at(i):
        return dst_all.at[pl.ds(i * CHUNK, CHUNK)]

    with jax.named_scope("edge_loop"):
        pltpu.async_copy(z_sh.at[src_at(0)], rows0, sem0)
        pltpu.async_copy(z_sh.at[src_at(1)], rows1, sem1)

        @pl.loop(0, NCHUNK, step=2)
        def _chunk(i):
            for b in range(2):
                idx = i + b
                pltpu.make_async_copy(z_sh.at[src_at(idx)],
                                      bufs[b], sems[b]).wait()
                pltpu.sync_copy(bufs[b], agg_sh.at[dst_at(idx)], add=True)
                nxt = idx + 2

                @pl.when(nxt < NCHUNK)
                def _():
                    pltpu.async_copy(z_sh.at[src_at(nxt)], bufs[b], sems[b])

    with jax.named_scope("dump"):
        plsc.subcore_barrier()
        pltpu.sync_copy(agg_sh.at[pl.ds(row_base, ROWS_PER_TILE)],
                        out_hbm.at[cid, pl.ds(row_base, ROWS_PER_TILE)])


_sc_edge = functools.partial(
    pl.kernel,
    out_type=jax.ShapeDtypeStruct((NC, N_PAD, D), jnp.bfloat16),
    compiler_params=pltpu.CompilerParams(use_tc_tiling_on_sc=False),
    mesh=plsc.VectorSubcoreMesh(
        core_axis_name="c", subcore_axis_name="s",
        num_cores=NC, num_subcores=NS),
    scratch_types=[
        pltpu.VMEM((EDGES_PER_TILE,), jnp.int32),
        pltpu.VMEM((EDGES_PER_TILE,), jnp.int32),
        pltpu.VMEM((CHUNK, D), jnp.bfloat16),
        pltpu.VMEM((CHUNK, D), jnp.bfloat16),
        pltpu.VMEM_SHARED((N_NODES, D), jnp.bfloat16),
        pltpu.VMEM_SHARED((N_PAD, D), jnp.bfloat16),
        pltpu.SemaphoreType.DMA,
        pltpu.SemaphoreType.DMA,
    ],
)(_sc_edge_body)


def kernel(node_feats, edge_index, W_msg, b_msg, W_out, b_out):
    edges = edge_index.astype(jnp.int32).reshape(2 * N_EDGES)
    z = _mm_relu(node_feats, W_msg, b_msg)
    zeros = jnp.zeros((ROWS_PER_TILE, D), jnp.bfloat16)
    partials = _sc_edge(edges, z, zeros)
    return _final_mm(partials, W_out, b_out)


# speedup vs baseline: 1.1885x; 1.1885x over previous
import functools

import jax
import jax.numpy as jnp
from jax import lax
from jax.experimental import pallas as pl
from jax.experimental.pallas import tpu as pltpu
from jax.experimental.pallas import tpu_sc as plsc

N_NODES = 10000
N_EDGES = 320000
D = 128

NC = 2
NS = 16
NW = NC * NS
CHUNK = 200
NCHUNK = 50
EDGES_PER_TILE = CHUNK * NCHUNK
N_PAD = 10240
ROWS_PER_TILE = N_PAD // NS

_BLK = 2000


def _mm_relu_body(x_ref, w_ref, b_ref, o_ref):
    z = jnp.dot(x_ref[...], w_ref[...], preferred_element_type=jnp.float32)
    o_ref[...] = jnp.maximum(z + b_ref[...], 0.0).astype(jnp.bfloat16)


def _mm_relu(x, w, b):
    n = x.shape[0]
    return pl.pallas_call(
        _mm_relu_body,
        grid=(n // _BLK,),
        in_specs=[
            pl.BlockSpec((_BLK, D), lambda i: (i, 0)),
            pl.BlockSpec((D, D), lambda i: (0, 0)),
            pl.BlockSpec((1, D), lambda i: (0, 0)),
        ],
        out_specs=pl.BlockSpec((_BLK, D), lambda i: (i, 0)),
        out_shape=jax.ShapeDtypeStruct((n, D), jnp.bfloat16),
    )(x, w, b.reshape(1, D))


def _final_mm_body(p_ref, w_ref, b_ref, o_ref):
    agg = (p_ref[0].astype(jnp.float32) + p_ref[1].astype(jnp.float32))
    o_ref[...] = (
        jnp.dot(agg, w_ref[...], preferred_element_type=jnp.float32)
        + b_ref[...])


def _final_mm(partials, w, b):
    n = N_NODES
    return pl.pallas_call(
        _final_mm_body,
        grid=(n // _BLK,),
        in_specs=[
            pl.BlockSpec((NC, _BLK, D), lambda i: (0, i, 0)),
            pl.BlockSpec((D, D), lambda i: (0, 0)),
            pl.BlockSpec((1, D), lambda i: (0, 0)),
        ],
        out_specs=pl.BlockSpec((_BLK, D), lambda i: (i, 0)),
        out_shape=jax.ShapeDtypeStruct((n, D), jnp.float32),
    )(partials, w, b.reshape(1, D))


def _sc_edge_body(edges_hbm, z_hbm, out_hbm,
                  src_all, dst_all, rows0, rows1, agg_sh, sem0, sem1):
    cid = lax.axis_index("c")
    sid = lax.axis_index("s")

    row_base = sid * ROWS_PER_TILE
    with jax.named_scope("init_zero_idx"):
        zvec = jnp.zeros((32,), jnp.bfloat16)

        @pl.loop(0, CHUNK)
        def _zero_row(r):
            for c in range(D // 32):
                rows0[r, pl.ds(c * 32, 32)] = zvec

        for r in range(0, ROWS_PER_TILE, CHUNK):
            nrows = min(ROWS_PER_TILE - r, CHUNK)
            pltpu.sync_copy(rows0.at[pl.ds(0, nrows)],
                            agg_sh.at[pl.ds(row_base + r, nrows)])

        wid = cid * NS + sid
        ebase = wid * EDGES_PER_TILE
        pltpu.sync_copy(edges_hbm.at[pl.ds(ebase, EDGES_PER_TILE)], src_all)
        pltpu.sync_copy(edges_hbm.at[pl.ds(N_EDGES + ebase, EDGES_PER_TILE)],
                        dst_all)
        plsc.subcore_barrier()

    bufs = (rows0, rows1)
    sems = (sem0, sem1)

    def src_at(i):
        return src_all.at[pl.ds(i * CHUNK, CHUNK)]

    def dst_at(i):
        return dst_all.at[pl.ds(i * CHUNK, CHUNK)]

    with jax.named_scope("edge_loop"):
        pltpu.async_copy(z_hbm.at[src_at(0)], rows0, sem0)
        pltpu.async_copy(z_hbm.at[src_at(1)], rows1, sem1)

        @pl.loop(0, NCHUNK, step=2)
        def _chunk(i):
            for b in range(2):
                idx = i + b
                pltpu.make_async_copy(z_hbm.at[src_at(idx)],
                                      bufs[b], sems[b]).wait()
                pltpu.sync_copy(bufs[b], agg_sh.at[dst_at(idx)], add=True)
                nxt = idx + 2

                @pl.when(nxt < NCHUNK)
                def _():
                    pltpu.async_copy(z_hbm.at[src_at(nxt)], bufs[b], sems[b])

    with jax.named_scope("dump"):
        plsc.subcore_barrier()
        pltpu.sync_copy(agg_sh.at[pl.ds(row_base, ROWS_PER_TILE)],
                        out_hbm.at[cid, pl.ds(row_base, ROWS_PER_TILE)])


_sc_edge = functools.partial(
    pl.kernel,
    out_type=jax.ShapeDtypeStruct((NC, N_PAD, D), jnp.bfloat16),
    compiler_params=pltpu.CompilerParams(use_tc_tiling_on_sc=False),
    mesh=plsc.VectorSubcoreMesh(
        core_axis_name="c", subcore_axis_name="s",
        num_cores=NC, num_subcores=NS),
    scratch_types=[
        pltpu.VMEM((EDGES_PER_TILE,), jnp.int32),
        pltpu.VMEM((EDGES_PER_TILE,), jnp.int32),
        pltpu.VMEM((CHUNK, D), jnp.bfloat16),
        pltpu.VMEM((CHUNK, D), jnp.bfloat16),
        pltpu.VMEM_SHARED((N_PAD, D), jnp.bfloat16),
        pltpu.SemaphoreType.DMA,
        pltpu.SemaphoreType.DMA,
    ],
)(_sc_edge_body)


def kernel(node_feats, edge_index, W_msg, b_msg, W_out, b_out):
    edges = edge_index.astype(jnp.int32).reshape(2 * N_EDGES)
    z = _mm_relu(node_feats, W_msg, b_msg)
    partials = _sc_edge(edges, z)
    return _final_mm(partials, W_out, b_out)
